# baseline (speedup 1.0000x reference)
import functools

import jax
import jax.numpy as jnp
from jax import lax
from jax.experimental import pallas as pl
from jax.experimental.pallas import tpu as pltpu
from jax.experimental.pallas import tpu_sc as plsc

_LANES = 16
_NC = 2
_NS = 16
_NW = _NC * _NS
_IDS_UNROLL = 4
_VB = 2048


def _sc_histogram(n: int, wlen: int, vpad: int):
    mesh = plsc.VectorSubcoreMesh(core_axis_name="c", subcore_axis_name="s")
    vrows = vpad // 128

    @functools.partial(
        pl.kernel,
        out_type=jax.ShapeDtypeStruct((_NC, vrows, 128), jnp.int32),
        mesh=mesh,
        compiler_params=pltpu.CompilerParams(needs_layout_passes=False),
        scratch_types=[
            pltpu.VMEM((wlen,), jnp.int32),
            pltpu.VMEM((vrows, 128), jnp.int32),
            pltpu.VMEM((3, 80), jnp.int32),
            pltpu.VMEM_SHARED((vrows, 128), jnp.int32),
        ],
    )
    def sc_body(ids_hbm, out_hbm, idx_v, hist_v, rowidx_v, shared_h):
        cid = lax.axis_index("c")
        sid = lax.axis_index("s")
        wid = sid * _NC + cid
        start = (wid * n) // _NW
        end = ((wid + 1) * n) // _NW
        astart = jnp.minimum((start // 8) * 8, n - wlen)
        pltpu.sync_copy(ids_hbm.at[pl.ds(astart, wlen)], idx_v)

        zero = jnp.zeros((_LANES,), jnp.int32)

        def zero_body(r, _):
            for u in range(128 // _LANES):
                hist_v[r, pl.ds(u * _LANES, _LANES)] = zero
            return 0

        lax.fori_loop(0, vrows, zero_body, 0)

        ones = jnp.ones((_LANES,), jnp.int32)
        lane = lax.iota(jnp.int32, _LANES)

        def scatter16(ids16, m=None):
            row16 = lax.shift_right_logical(ids16, 7)
            col16 = jnp.bitwise_and(ids16, 127)
            plsc.addupdate_scatter(hist_v, [row16, col16], ones, mask=m)

        n_groups = wlen // _LANES
        g_lo = (start - astart + _LANES - 1) // _LANES
        g_hi = (end - astart) // _LANES

        def edge_body(g, _):
            pos = astart + g * _LANES + lane
            m = jnp.logical_and(pos >= start, pos < end)
            scatter16(idx_v[pl.ds(g * _LANES, _LANES)], m)
            return 0

        for j in range(3):
            for i in range(80 // _LANES):
                rowidx_v[j, pl.ds(i * _LANES, _LANES)] = (
                    lane + (j * 80 + i * _LANES))

        @pl.when(sid == 0)
        def _zero_shared():
            pltpu.sync_copy(hist_v, shared_h)

        plsc.subcore_barrier()

        lax.fori_loop(0, g_lo, edge_body, 0)
        n_mid = (g_hi - g_lo) // _IDS_UNROLL

        def mid_strided(i, _):
            for u in range(_IDS_UNROLL):
                g = g_lo + i * _IDS_UNROLL + u
                scatter16(idx_v[pl.ds(g * _LANES, _LANES)])
            return 0

        lax.fori_loop(0, n_mid, mid_strided, 0)
        lax.fori_loop(g_lo + n_mid * _IDS_UNROLL, n_groups, edge_body, 0)

        for j in range(3):
            pltpu.sync_copy(hist_v.at[pl.ds(j * 80, 80)],
                            shared_h.at[rowidx_v.at[j]], add=True)
        plsc.subcore_barrier()

        @pl.when(sid < vrows // 16)
        def _writeback():
            pltpu.sync_copy(shared_h.at[pl.ds(sid * 16, 16)],
                            out_hbm.at[cid, pl.ds(sid * 16, 16)])

    return sc_body


def _tc_contract_body(h_ref, t_ref, wg_ref, bg_ref, wo_ref, bo_ref,
                      o_ref, acc_ref, *, vocab, n_rows, n_blocks):
    k = pl.program_id(0)

    @pl.when(k == 0)
    def _init():
        acc_ref[...] = jnp.zeros_like(acc_ref)

    counts = jnp.sum(h_ref[...], axis=0)
    c = counts.astype(jnp.bfloat16).reshape(1, -1)
    blk = t_ref.shape[0]

    @pl.when(k < n_blocks - 1)
    def _full():
        acc_ref[...] += jnp.dot(c, t_ref[...],
                                preferred_element_type=jnp.float32)

    @pl.when(k == n_blocks - 1)
    def _masked():
        row = lax.broadcasted_iota(jnp.int32, t_ref.shape, 0) + k * blk
        tb = jnp.where(row < vocab, t_ref[...], jnp.bfloat16(0))
        acc_ref[...] += jnp.dot(c, tb, preferred_element_type=jnp.float32)

    @pl.when(k == n_blocks - 1)
    def _finish():
        m = acc_ref[...] * jnp.float32(1.0 / n_rows)
        h = jnp.dot(m, wg_ref[...], preferred_element_type=jnp.float32,
                    precision=lax.Precision.HIGHEST) + bg_ref[...]
        o = jnp.dot(h, wo_ref[...], preferred_element_type=jnp.float32,
                    precision=lax.Precision.HIGHEST) + bo_ref[...]
        o_ref[...] = o.reshape(o_ref.shape)


def kernel(input_ids, embed_table, W_gnn, b_gnn, W_out, b_out):
    n = input_ids.shape[0]
    vocab, hidden = embed_table.shape
    out_dim = W_out.shape[1]
    max_span = -(-n // _NW)
    wlen = -(-(max_span + 7) // _LANES) * _LANES
    vpad = -(-vocab // _VB) * _VB
    n_blocks = vpad // _VB

    ids = input_ids.astype(jnp.int32)
    hist = _sc_histogram(n, wlen, vpad)(ids)

    def _conv_body(t_ref, o_ref):
        o_ref[...] = t_ref[...].astype(jnp.bfloat16)

    table_bf16 = pl.pallas_call(
        _conv_body,
        grid=(n_blocks,),
        in_specs=[pl.BlockSpec((_VB, hidden), lambda k: (k, 0))],
        out_specs=pl.BlockSpec((_VB, hidden), lambda k: (k, 0)),
        out_shape=jax.ShapeDtypeStruct((vocab, hidden), jnp.bfloat16),
    )(embed_table)

    out = pl.pallas_call(
        functools.partial(_tc_contract_body, vocab=vocab, n_rows=n,
                          n_blocks=n_blocks),
        grid=(n_blocks,),
        in_specs=[
            pl.BlockSpec((_NC, _VB // 128, 128), lambda k: (0, k, 0)),
            pl.BlockSpec((_VB, hidden), lambda k: (k, 0)),
            pl.BlockSpec((hidden, hidden), lambda k: (0, 0)),
            pl.BlockSpec((1, hidden), lambda k: (0, 0)),
            pl.BlockSpec((hidden, out_dim), lambda k: (0, 0)),
            pl.BlockSpec((1, out_dim), lambda k: (0, 0)),
        ],
        out_specs=pl.BlockSpec((out_dim,), lambda k: (0,)),
        out_shape=jax.ShapeDtypeStruct((out_dim,), jnp.float32),
        scratch_shapes=[pltpu.VMEM((1, hidden), jnp.float32)],
    )(hist, table_bf16, W_gnn, b_gnn.reshape(1, hidden), W_out,
      b_out.reshape(1, out_dim))
    return out


# speedup vs baseline: 1.2791x; 1.2791x over previous
import functools

import jax
import jax.numpy as jnp
from jax import lax
from jax.experimental import pallas as pl
from jax.experimental.pallas import tpu as pltpu
from jax.experimental.pallas import tpu_sc as plsc

_LANES = 16
_NC = 2
_NS = 16
_NW = _NC * _NS
_IDS_UNROLL = 4
_VB = 6144


def _sc_histogram(n: int, wlen: int, vpad: int):
    mesh = plsc.VectorSubcoreMesh(core_axis_name="c", subcore_axis_name="s")
    vrows = vpad // 128

    @functools.partial(
        pl.kernel,
        out_type=jax.ShapeDtypeStruct((_NC, vrows, 128), jnp.int32),
        mesh=mesh,
        compiler_params=pltpu.CompilerParams(needs_layout_passes=False),
        scratch_types=[
            pltpu.VMEM((wlen,), jnp.int32),
            pltpu.VMEM((vrows, 128), jnp.int32),
            pltpu.VMEM((3, 80), jnp.int32),
            pltpu.VMEM_SHARED((vrows, 128), jnp.int32),
        ],
    )
    def sc_body(ids_hbm, out_hbm, idx_v, hist_v, rowidx_v, shared_h):
        cid = lax.axis_index("c")
        sid = lax.axis_index("s")
        wid = sid * _NC + cid
        start = (wid * n) // _NW
        end = ((wid + 1) * n) // _NW
        astart = jnp.minimum((start // 8) * 8, n - wlen)
        pltpu.sync_copy(ids_hbm.at[pl.ds(astart, wlen)], idx_v)

        zero = jnp.zeros((_LANES,), jnp.int32)

        def zero_body(r, _):
            for u in range(128 // _LANES):
                hist_v[r, pl.ds(u * _LANES, _LANES)] = zero
            return 0

        lax.fori_loop(0, vrows, zero_body, 0)

        ones = jnp.ones((_LANES,), jnp.int32)
        lane = lax.iota(jnp.int32, _LANES)

        def scatter16(ids16, m=None):
            row16 = lax.shift_right_logical(ids16, 7)
            col16 = jnp.bitwise_and(ids16, 127)
            plsc.addupdate_scatter(hist_v, [row16, col16], ones, mask=m)

        n_groups = wlen // _LANES
        g_lo = (start - astart + _LANES - 1) // _LANES
        g_hi = (end - astart) // _LANES

        def edge_body(g, _):
            pos = astart + g * _LANES + lane
            m = jnp.logical_and(pos >= start, pos < end)
            scatter16(idx_v[pl.ds(g * _LANES, _LANES)], m)
            return 0

        for j in range(3):
            for i in range(80 // _LANES):
                rowidx_v[j, pl.ds(i * _LANES, _LANES)] = (
                    lane + (j * 80 + i * _LANES))

        @pl.when(sid == 0)
        def _zero_shared():
            pltpu.sync_copy(hist_v, shared_h)

        plsc.subcore_barrier()

        lax.fori_loop(0, g_lo, edge_body, 0)
        n_mid = (g_hi - g_lo) // _IDS_UNROLL

        def mid_strided(i, _):
            for u in range(_IDS_UNROLL):
                g = g_lo + i * _IDS_UNROLL + u
                scatter16(idx_v[pl.ds(g * _LANES, _LANES)])
            return 0

        lax.fori_loop(0, n_mid, mid_strided, 0)
        lax.fori_loop(g_lo + n_mid * _IDS_UNROLL, n_groups, edge_body, 0)

        for j in range(3):
            pltpu.sync_copy(hist_v.at[pl.ds(j * 80, 80)],
                            shared_h.at[rowidx_v.at[j]], add=True)
        plsc.subcore_barrier()

        @pl.when(sid < vrows // 16)
        def _writeback():
            pltpu.sync_copy(shared_h.at[pl.ds(sid * 16, 16)],
                            out_hbm.at[cid, pl.ds(sid * 16, 16)])

    return sc_body


def _tc_contract_body(h_ref, t_ref, wg_ref, bg_ref, wo_ref, bo_ref,
                      o_ref, acc_ref, *, vocab, n_rows, n_blocks):
    k = pl.program_id(0)

    @pl.when(k == 0)
    def _init():
        acc_ref[...] = jnp.zeros_like(acc_ref)

    counts = jnp.sum(h_ref[...], axis=0)
    c = counts.astype(jnp.bfloat16).reshape(1, -1)
    blk = t_ref.shape[0]

    @pl.when(k < n_blocks - 1)
    def _full():
        acc_ref[...] += jnp.dot(c, t_ref[...],
                                preferred_element_type=jnp.float32)

    @pl.when(k == n_blocks - 1)
    def _masked():
        row = lax.broadcasted_iota(jnp.int32, t_ref.shape, 0) + k * blk
        tb = jnp.where(row < vocab, t_ref[...], 0.0)
        acc_ref[...] += jnp.dot(c, tb, preferred_element_type=jnp.float32)

    @pl.when(k == n_blocks - 1)
    def _finish():
        m = acc_ref[...] * jnp.float32(1.0 / n_rows)
        h = jnp.dot(m, wg_ref[...], preferred_element_type=jnp.float32,
                    precision=lax.Precision.HIGHEST) + bg_ref[...]
        o = jnp.dot(h, wo_ref[...], preferred_element_type=jnp.float32,
                    precision=lax.Precision.HIGHEST) + bo_ref[...]
        o_ref[...] = o.reshape(o_ref.shape)


def kernel(input_ids, embed_table, W_gnn, b_gnn, W_out, b_out):
    n = input_ids.shape[0]
    vocab, hidden = embed_table.shape
    out_dim = W_out.shape[1]
    max_span = -(-n // _NW)
    wlen = -(-(max_span + 7) // _LANES) * _LANES
    vpad = -(-vocab // _VB) * _VB
    n_blocks = vpad // _VB

    ids = input_ids.astype(jnp.int32)
    hist = _sc_histogram(n, wlen, vpad)(ids)

    out = pl.pallas_call(
        functools.partial(_tc_contract_body, vocab=vocab, n_rows=n,
                          n_blocks=n_blocks),
        grid=(n_blocks,),
        in_specs=[
            pl.BlockSpec((_NC, _VB // 128, 128), lambda k: (0, k, 0)),
            pl.BlockSpec((_VB, hidden), lambda k: (k, 0)),
            pl.BlockSpec((hidden, hidden), lambda k: (0, 0)),
            pl.BlockSpec((1, hidden), lambda k: (0, 0)),
            pl.BlockSpec((hidden, out_dim), lambda k: (0, 0)),
            pl.BlockSpec((1, out_dim), lambda k: (0, 0)),
        ],
        out_specs=pl.BlockSpec((out_dim,), lambda k: (0,)),
        out_shape=jax.ShapeDtypeStruct((out_dim,), jnp.float32),
        scratch_shapes=[pltpu.VMEM((1, hidden), jnp.float32)],
    )(hist, embed_table, W_gnn, b_gnn.reshape(1, hidden), W_out,
      b_out.reshape(1, out_dim))
    return out


# speedup vs baseline: 1.3015x; 1.0175x over previous
import functools

import jax
import jax.numpy as jnp
from jax import lax
from jax.experimental import pallas as pl
from jax.experimental.pallas import tpu as pltpu
from jax.experimental.pallas import tpu_sc as plsc

_LANES = 16
_NC = 2
_NS = 16
_NW = _NC * _NS
_IDS_UNROLL = 4
_VB = 10240


def _sc_histogram(n: int, wlen: int, vpad: int):
    mesh = plsc.VectorSubcoreMesh(core_axis_name="c", subcore_axis_name="s")
    vrows = vpad // 128

    @functools.partial(
        pl.kernel,
        out_type=jax.ShapeDtypeStruct((_NC, vrows, 128), jnp.int32),
        mesh=mesh,
        compiler_params=pltpu.CompilerParams(needs_layout_passes=False),
        scratch_types=[
            pltpu.VMEM((wlen,), jnp.int32),
            pltpu.VMEM((vrows, 128), jnp.int32),
            pltpu.VMEM((3, 80), jnp.int32),
            pltpu.VMEM_SHARED((vrows, 128), jnp.int32),
        ],
    )
    def sc_body(ids_hbm, out_hbm, idx_v, hist_v, rowidx_v, shared_h):
        cid = lax.axis_index("c")
        sid = lax.axis_index("s")
        wid = sid * _NC + cid
        start = (wid * n) // _NW
        end = ((wid + 1) * n) // _NW
        astart = jnp.minimum((start // 8) * 8, n - wlen)
        pltpu.sync_copy(ids_hbm.at[pl.ds(astart, wlen)], idx_v)

        zero = jnp.zeros((_LANES,), jnp.int32)

        def zero_body(r, _):
            for u in range(128 // _LANES):
                hist_v[r, pl.ds(u * _LANES, _LANES)] = zero
            return 0

        lax.fori_loop(0, vrows, zero_body, 0)

        ones = jnp.ones((_LANES,), jnp.int32)
        lane = lax.iota(jnp.int32, _LANES)

        def scatter16(ids16, m=None):
            row16 = lax.shift_right_logical(ids16, 7)
            col16 = jnp.bitwise_and(ids16, 127)
            plsc.addupdate_scatter(hist_v, [row16, col16], ones, mask=m)

        n_groups = wlen // _LANES
        g_lo = (start - astart + _LANES - 1) // _LANES
        g_hi = (end - astart) // _LANES

        def edge_body(g, _):
            pos = astart + g * _LANES + lane
            m = jnp.logical_and(pos >= start, pos < end)
            scatter16(idx_v[pl.ds(g * _LANES, _LANES)], m)
            return 0

        for j in range(3):
            for i in range(80 // _LANES):
                rowidx_v[j, pl.ds(i * _LANES, _LANES)] = (
                    lane + (j * 80 + i * _LANES))

        @pl.when(sid == 0)
        def _zero_shared():
            pltpu.sync_copy(hist_v, shared_h)

        plsc.subcore_barrier()

        lax.fori_loop(0, g_lo, edge_body, 0)
        n_mid = (g_hi - g_lo) // _IDS_UNROLL

        def mid_strided(i, _):
            for u in range(_IDS_UNROLL):
                g = g_lo + i * _IDS_UNROLL + u
                scatter16(idx_v[pl.ds(g * _LANES, _LANES)])
            return 0

        lax.fori_loop(0, n_mid, mid_strided, 0)
        lax.fori_loop(g_lo + n_mid * _IDS_UNROLL, n_groups, edge_body, 0)

        for j in range(3):
            pltpu.sync_copy(hist_v.at[pl.ds(j * 80, 80)],
                            shared_h.at[rowidx_v.at[j]], add=True)
        plsc.subcore_barrier()

        @pl.when(sid < vrows // 16)
        def _writeback():
            pltpu.sync_copy(shared_h.at[pl.ds(sid * 16, 16)],
                            out_hbm.at[cid, pl.ds(sid * 16, 16)])

    return sc_body


def _tc_contract_body(h_ref, t_ref, wg_ref, bg_ref, wo_ref, bo_ref,
                      o_ref, acc_ref, *, vocab, n_rows, n_blocks):
    k = pl.program_id(0)

    @pl.when(k == 0)
    def _init():
        acc_ref[...] = jnp.zeros_like(acc_ref)

    counts = jnp.sum(h_ref[...], axis=0)
    c = counts.astype(jnp.bfloat16).reshape(1, -1)
    blk = t_ref.shape[0]

    @pl.when(k < n_blocks - 1)
    def _full():
        acc_ref[...] += jnp.dot(c, t_ref[...],
                                preferred_element_type=jnp.float32)

    @pl.when(k == n_blocks - 1)
    def _masked():
        row = lax.broadcasted_iota(jnp.int32, t_ref.shape, 0) + k * blk
        tb = jnp.where(row < vocab, t_ref[...], 0.0)
        acc_ref[...] += jnp.dot(c, tb, preferred_element_type=jnp.float32)

    @pl.when(k == n_blocks - 1)
    def _finish():
        m = acc_ref[...] * jnp.float32(1.0 / n_rows)
        h = jnp.dot(m, wg_ref[...], preferred_element_type=jnp.float32,
                    precision=lax.Precision.HIGHEST) + bg_ref[...]
        o = jnp.dot(h, wo_ref[...], preferred_element_type=jnp.float32,
                    precision=lax.Precision.HIGHEST) + bo_ref[...]
        o_ref[...] = o.reshape(o_ref.shape)


def kernel(input_ids, embed_table, W_gnn, b_gnn, W_out, b_out):
    n = input_ids.shape[0]
    vocab, hidden = embed_table.shape
    out_dim = W_out.shape[1]
    max_span = -(-n // _NW)
    wlen = -(-(max_span + 7) // _LANES) * _LANES
    vpad = -(-vocab // _VB) * _VB
    n_blocks = vpad // _VB

    ids = input_ids.astype(jnp.int32)
    hist = _sc_histogram(n, wlen, vpad)(ids)

    out = pl.pallas_call(
        functools.partial(_tc_contract_body, vocab=vocab, n_rows=n,
                          n_blocks=n_blocks),
        grid=(n_blocks,),
        in_specs=[
            pl.BlockSpec((_NC, _VB // 128, 128), lambda k: (0, k, 0)),
            pl.BlockSpec((_VB, hidden), lambda k: (k, 0)),
            pl.BlockSpec((hidden, hidden), lambda k: (0, 0)),
            pl.BlockSpec((1, hidden), lambda k: (0, 0)),
            pl.BlockSpec((hidden, out_dim), lambda k: (0, 0)),
            pl.BlockSpec((1, out_dim), lambda k: (0, 0)),
        ],
        out_specs=pl.BlockSpec((out_dim,), lambda k: (0,)),
        out_shape=jax.ShapeDtypeStruct((out_dim,), jnp.float32),
        scratch_shapes=[pltpu.VMEM((1, hidden), jnp.float32)],
    )(hist, embed_table, W_gnn, b_gnn.reshape(1, hidden), W_out,
      b_out.reshape(1, out_dim))
    return out


# speedup vs baseline: 1.3289x; 1.0210x over previous
import functools

import jax
import jax.numpy as jnp
from jax import lax
from jax.experimental import pallas as pl
from jax.experimental.pallas import tpu as pltpu
from jax.experimental.pallas import tpu_sc as plsc

_LANES = 16
_NC = 2
_NS = 16
_NW = _NC * _NS
_IDS_UNROLL = 4
_VB = 10240


def _sc_histogram(n: int, wlen: int, vpad: int):
    mesh = plsc.VectorSubcoreMesh(core_axis_name="c", subcore_axis_name="s")
    vrows = vpad // 128

    @functools.partial(
        pl.kernel,
        out_type=jax.ShapeDtypeStruct((_NW, vrows, 128), jnp.int32),
        mesh=mesh,
        compiler_params=pltpu.CompilerParams(needs_layout_passes=False),
        scratch_types=[
            pltpu.VMEM((wlen,), jnp.int32),
            pltpu.VMEM((vrows, 128), jnp.int32),
        ],
    )
    def sc_body(ids_hbm, out_hbm, idx_v, hist_v):
        wid = lax.axis_index("s") * _NC + lax.axis_index("c")
        start = (wid * n) // _NW
        end = ((wid + 1) * n) // _NW
        astart = jnp.minimum((start // 8) * 8, n - wlen)
        pltpu.sync_copy(ids_hbm.at[pl.ds(astart, wlen)], idx_v)

        zero = jnp.zeros((_LANES,), jnp.int32)

        def zero_body(r, _):
            for u in range(128 // _LANES):
                hist_v[r, pl.ds(u * _LANES, _LANES)] = zero
            return 0

        lax.fori_loop(0, vrows, zero_body, 0)

        ones = jnp.ones((_LANES,), jnp.int32)
        lane = lax.iota(jnp.int32, _LANES)

        def scatter16(ids16, m=None):
            row16 = lax.shift_right_logical(ids16, 7)
            col16 = jnp.bitwise_and(ids16, 127)
            plsc.addupdate_scatter(hist_v, [row16, col16], ones, mask=m)

        n_groups = wlen // _LANES
        g_lo = (start - astart + _LANES - 1) // _LANES
        g_hi = (end - astart) // _LANES

        def edge_body(g, _):
            pos = astart + g * _LANES + lane
            m = jnp.logical_and(pos >= start, pos < end)
            scatter16(idx_v[pl.ds(g * _LANES, _LANES)], m)
            return 0

        lax.fori_loop(0, g_lo, edge_body, 0)
        n_mid = (g_hi - g_lo) // _IDS_UNROLL

        def mid_strided(i, _):
            for u in range(_IDS_UNROLL):
                g = g_lo + i * _IDS_UNROLL + u
                scatter16(idx_v[pl.ds(g * _LANES, _LANES)])
            return 0

        lax.fori_loop(0, n_mid, mid_strided, 0)
        lax.fori_loop(g_lo + n_mid * _IDS_UNROLL, n_groups, edge_body, 0)
        pltpu.sync_copy(hist_v, out_hbm.at[wid])

    return sc_body


def _tc_contract_body(h_ref, t_ref, wg_ref, bg_ref, wo_ref, bo_ref,
                      o_ref, acc_ref, *, vocab, n_rows, n_blocks):
    k = pl.program_id(0)

    @pl.when(k == 0)
    def _init():
        acc_ref[...] = jnp.zeros_like(acc_ref)

    counts = jnp.sum(h_ref[...], axis=0)
    c = counts.astype(jnp.bfloat16).reshape(1, -1)
    blk = t_ref.shape[0]

    @pl.when(k < n_blocks - 1)
    def _full():
        acc_ref[...] += jnp.dot(c, t_ref[...].astype(jnp.bfloat16),
                                preferred_element_type=jnp.float32)

    @pl.when(k == n_blocks - 1)
    def _masked():
        row = lax.broadcasted_iota(jnp.int32, t_ref.shape, 0) + k * blk
        tb = jnp.where(row < vocab, t_ref[...], 0.0).astype(jnp.bfloat16)
        acc_ref[...] += jnp.dot(c, tb, preferred_element_type=jnp.float32)

    @pl.when(k == n_blocks - 1)
    def _finish():
        m = acc_ref[...] * jnp.float32(1.0 / n_rows)
        h = jnp.dot(m, wg_ref[...], preferred_element_type=jnp.float32,
                    precision=lax.Precision.HIGHEST) + bg_ref[...]
        o = jnp.dot(h, wo_ref[...], preferred_element_type=jnp.float32,
                    precision=lax.Precision.HIGHEST) + bo_ref[...]
        o_ref[...] = o.reshape(o_ref.shape)


def kernel(input_ids, embed_table, W_gnn, b_gnn, W_out, b_out):
    n = input_ids.shape[0]
    vocab, hidden = embed_table.shape
    out_dim = W_out.shape[1]
    max_span = -(-n // _NW)
    wlen = -(-(max_span + 7) // _LANES) * _LANES
    vpad = -(-vocab // _VB) * _VB
    n_blocks = vpad // _VB

    ids = input_ids.astype(jnp.int32)
    hist = _sc_histogram(n, wlen, vpad)(ids)

    out = pl.pallas_call(
        functools.partial(_tc_contract_body, vocab=vocab, n_rows=n,
                          n_blocks=n_blocks),
        grid=(n_blocks,),
        in_specs=[
            pl.BlockSpec((_NW, _VB // 128, 128), lambda k: (0, k, 0)),
            pl.BlockSpec((_VB, hidden), lambda k: (k, 0)),
            pl.BlockSpec((hidden, hidden), lambda k: (0, 0)),
            pl.BlockSpec((1, hidden), lambda k: (0, 0)),
            pl.BlockSpec((hidden, out_dim), lambda k: (0, 0)),
            pl.BlockSpec((1, out_dim), lambda k: (0, 0)),
        ],
        out_specs=pl.BlockSpec((out_dim,), lambda k: (0,)),
        out_shape=jax.ShapeDtypeStruct((out_dim,), jnp.float32),
        scratch_shapes=[pltpu.VMEM((1, hidden), jnp.float32)],
    )(hist, embed_table, W_gnn, b_gnn.reshape(1, hidden), W_out,
      b_out.reshape(1, out_dim))
    return out
